# baseline (speedup 1.0000x reference)
import functools

import jax
import jax.numpy as jnp
from jax import lax
from jax.experimental import pallas as pl
from jax.experimental.pallas import tpu as pltpu
from jax.experimental.pallas import tpu_sc as plsc

_LANES = 16
_BB = 128


@functools.lru_cache(maxsize=None)
def _make_sc_encoder(batch, seq_len, d_model, pos_rows, vocab):
    info = plsc.get_sparse_core_info()
    nc, ns = info.num_cores, info.num_subcores
    nw = nc * ns
    assert batch == nw * _BB
    assert d_model % _LANES == 0 and seq_len % 2 == 0
    lgroups = _BB // _LANES

    mesh = plsc.VectorSubcoreMesh(core_axis_name="c", subcore_axis_name="s")

    scratch = [
        pltpu.VMEM((seq_len, _BB), jnp.int32),
        pltpu.VMEM((pos_rows, d_model), jnp.float32),
    ]
    scratch[1] = pltpu.VMEM((d_model, pos_rows), jnp.float32)
    scratch += [pltpu.VMEM((_BB,), jnp.int32) for _ in range(2)]
    scratch += [pltpu.VMEM((_BB, 2 * d_model), jnp.float32) for _ in range(2)]
    scratch += [pltpu.VMEM((d_model, _BB), jnp.float32) for _ in range(2)]
    scratch += [pltpu.SemaphoreType.DMA for _ in range(4)]

    @functools.partial(
        pl.kernel,
        mesh=mesh,
        out_type=jax.ShapeDtypeStruct((seq_len, d_model, batch), jnp.float32),
        scratch_types=scratch,
        compiler_params=pltpu.CompilerParams(
            use_tc_tiling_on_sc=True, needs_layout_passes=False),
    )
    def enc(tok_hbm, tbl_hbm, pos_hbm, out_hbm, idx_v, pos_v,
            pidx0, pidx1, bin0, bin1, bout0, bout1, gsem0, gsem1, ssem0, ssem1):
        pidx = (pidx0, pidx1)
        bins = (bin0, bin1)
        bouts = (bout0, bout1)
        gsems = (gsem0, gsem1)
        ssems = (ssem0, ssem1)

        wid = lax.axis_index("s") * nc + lax.axis_index("c")
        b0 = wid * _BB

        pltpu.sync_copy(pos_hbm, pos_v)
        pltpu.sync_copy(tok_hbm.at[:, pl.ds(b0, _BB)], idx_v)

        iotas = [lax.iota(jnp.int32, _LANES) + l * _LANES for l in range(lgroups)]

        def build_pairs(s, bb):
            for l in range(lgroups):
                sl = pl.ds(l * _LANES, _LANES)
                pidx[bb][sl] = lax.shift_right_logical(idx_v[s, sl], 1)

        def fire_gather(bb):
            pltpu.async_copy(tbl_hbm.at[pidx[bb]], bins[bb], gsems[bb])

        def wait_gather(bb):
            pltpu.make_async_copy(tbl_hbm.at[pidx[bb]], bins[bb], gsems[bb]).wait()

        def fire_scatter(s, bb):
            pltpu.async_copy(bouts[bb], out_hbm.at[s, :, pl.ds(b0, _BB)], ssems[bb])

        def wait_scatter(bb):
            pltpu.make_async_copy(
                bouts[bb], out_hbm.at[0, :, pl.ds(b0, _BB)], ssems[bb]).wait()

        def compute(s, bb):
            colbs = []
            for l in range(lgroups):
                sl = pl.ds(l * _LANES, _LANES)
                colbs.append(
                    lax.shift_left(lax.bitwise_and(idx_v[s, sl], 1), 6))

            def dbody(i, carry):
                for dd in range(4):
                    d = i * 4 + dd
                    pv = pos_v[d, pl.ds(s, _LANES)]
                    p = jnp.full((_LANES,), pv[0], dtype=jnp.float32)
                    for l in range(lgroups):
                        v = plsc.load_gather(bins[bb], [iotas[l], colbs[l] + d])
                        bouts[bb][d, pl.ds(l * _LANES, _LANES)] = v + p
                return carry

            lax.fori_loop(0, d_model // 4, dbody, 0)

        for s in range(2):
            build_pairs(s, s)
            fire_gather(s)
        for s in range(2):
            bb = s
            wait_gather(bb)
            compute(s, bb)
            fire_scatter(s, bb)
            build_pairs(s + 2, bb)
            fire_gather(bb)

        def body(o, carry):
            for bb in range(2):
                s = 2 + o * 2 + bb
                wait_gather(bb)
                wait_scatter(bb)
                compute(s, bb)
                fire_scatter(s, bb)
                build_pairs(s + 2, bb)
                fire_gather(bb)
            return carry

        lax.fori_loop(0, (seq_len - 4) // 2, body, 0)

        for s in range(seq_len - 2, seq_len):
            bb = s % 2
            wait_gather(bb)
            wait_scatter(bb)
            compute(s, bb)
            fire_scatter(s, bb)
        for bb in range(2):
            wait_scatter(bb)

    return enc


def kernel(token_ids, token_embed, pos_embed):
    b, s = token_ids.shape
    vocab, d = token_embed.shape
    pos_rows = pos_embed.shape[0]
    tok_t = token_ids.T.astype(jnp.int32)
    pos_t = pos_embed.T
    tbl2 = token_embed.reshape(vocab // 2, 2 * d)
    enc = _make_sc_encoder(b, s, d, pos_rows, vocab)
    out_phys = enc(tok_t, tbl2, pos_t)
    return jnp.transpose(out_phys, (2, 0, 1))
